# baseline (speedup 1.0000x reference)
import functools

import jax
import jax.numpy as jnp
from jax.experimental import pallas as pl
from jax.experimental.pallas import tpu as pltpu


_N_HALF = 4096
_N_PROJ = 8192
_N_PC = 8192
_D = 128
_DA = _D + 2

_BI = 512
_BJ = 2048
_NI = _N_PROJ // _BI
_NJ = _N_PC // _BJ
_NI_HALF = _N_HALF // _BI


def _build_keys_kernel(pc_ref, pca_ref):
    pc = pc_ref[...]
    bb = jnp.sum(pc * pc, axis=1, keepdims=True)
    bb_hi = bb.astype(jnp.bfloat16)
    bb_lo = (bb - bb_hi.astype(jnp.float32)).astype(jnp.bfloat16)
    pca_ref[:, :_D] = pc.astype(jnp.bfloat16)
    pca_ref[:, _D:_D + 1] = bb_hi
    pca_ref[:, _D + 1:] = bb_lo


def _dist_loss_kernel(z_ref, g_ref, pca_ref, ze_ref, ge_ref, mp_ref,
                      ft_sum_ref, mp_sum_ref, rhs_scr):
    i = pl.program_id(0)
    first_half = i < _NI_HALF

    a = jnp.where(first_half, z_ref[...], g_ref[...])
    at = jnp.swapaxes(a, 0, 1)
    aa = jnp.sum(at * at, axis=0, keepdims=True)

    rhs_scr[:_D, :] = (-2.0 * at).astype(jnp.bfloat16)
    rhs_scr[_D:, :] = jnp.ones((2, _BI), jnp.bfloat16)
    rhs = rhs_scr[...]

    pm = None
    for j in range(_NJ):
        ab = jax.lax.dot_general(
            pca_ref[j * _BJ:(j + 1) * _BJ, :], rhs,
            dimension_numbers=(((1,), (0,)), ((), ())),
            preferred_element_type=jnp.float32)
        m = jnp.min(ab, axis=0, keepdims=True)
        pm = m if pm is None else jnp.minimum(pm, m)

    d = pm + aa
    pe = jnp.where(first_half, ze_ref[...], ge_ref[...])
    pet = jnp.swapaxes(pe, 0, 1)
    ft = jnp.abs(jnp.sqrt(jnp.abs(d) + 1e-7) - jnp.abs(pet))
    ft_blk = jnp.sum(ft)
    mp_blk = jnp.sum(jnp.abs(mp_ref[...]))

    @pl.when(i == 0)
    def _():
        ft_sum_ref[0, 0] = ft_blk
        mp_sum_ref[0, 0] = mp_blk

    @pl.when(i > 0)
    def _():
        ft_sum_ref[0, 0] += ft_blk
        mp_sum_ref[0, 0] += mp_blk


def _half_map(i):
    return (jnp.minimum(i, _NI_HALF - 1), 0)


def _gen_map(i):
    return (jnp.maximum(i - _NI_HALF, 0), 0)


@functools.partial(jax.jit, static_argnames=("interpret",))
def _dist_loss(zero_pts, gen_pts, pc, zero_eval, gen_eval, manifold,
               interpret=False):
    pca = pl.pallas_call(
        _build_keys_kernel,
        out_shape=jax.ShapeDtypeStruct((_N_PC, _DA), jnp.bfloat16),
        interpret=interpret,
    )(pc)

    ft_sum, mp_sum = pl.pallas_call(
        _dist_loss_kernel,
        grid=(_NI,),
        in_specs=[
            pl.BlockSpec((_BI, _D), _half_map),
            pl.BlockSpec((_BI, _D), _gen_map),
            pl.BlockSpec((_N_PC, _DA), lambda i: (0, 0)),
            pl.BlockSpec((_BI, 1), _half_map),
            pl.BlockSpec((_BI, 1), _gen_map),
            pl.BlockSpec((_BI, 1), lambda i: (i, 0)),
        ],
        out_specs=[
            pl.BlockSpec(memory_space=pltpu.SMEM),
            pl.BlockSpec(memory_space=pltpu.SMEM),
        ],
        out_shape=[
            jax.ShapeDtypeStruct((1, 1), jnp.float32),
            jax.ShapeDtypeStruct((1, 1), jnp.float32),
        ],
        scratch_shapes=[
            pltpu.VMEM((_DA, _BI), jnp.bfloat16),
        ],
        compiler_params=pltpu.CompilerParams(
            dimension_semantics=("arbitrary",),
        ),
        interpret=interpret,
    )(zero_pts, gen_pts, pca, zero_eval, gen_eval, manifold)
    return ft_sum[0, 0], mp_sum[0, 0]


def kernel(zerolevelset_points, genlevelset_points, pc_input,
           zerolevelset_eval, gen_points_eval, manifold_pnts_pred,
           loss_lambda):
    ft_sum, mp_sum = _dist_loss(
        zerolevelset_points, genlevelset_points, pc_input,
        zerolevelset_eval, gen_points_eval, manifold_pnts_pred)

    mean_first = ft_sum / _N_PROJ
    mean_second = mp_sum / _N_PROJ
    ll = 0.1 if loss_lambda is None else loss_lambda
    loss = mean_first + ll * mean_second
    return (loss, mean_first, mean_second)


# speedup vs baseline: 1.0809x; 1.0809x over previous
import functools

import jax
import jax.numpy as jnp
from jax.experimental import pallas as pl
from jax.experimental.pallas import tpu as pltpu


_N_HALF = 4096
_N_PROJ = 8192
_N_PC = 8192
_D = 128
_DA = _D + 2

_BI = 2048
_BJ = 8192
_NI = _N_PROJ // _BI
_NJ = _N_PC // _BJ
_NI_HALF = _N_HALF // _BI


def _build_keys_kernel(pc_ref, pca_ref):
    pc = pc_ref[...]
    bb = jnp.sum(pc * pc, axis=1, keepdims=True)
    bb_hi = bb.astype(jnp.bfloat16)
    bb_lo = (bb - bb_hi.astype(jnp.float32)).astype(jnp.bfloat16)
    pca_ref[:, :_D] = pc.astype(jnp.bfloat16)
    pca_ref[:, _D:_D + 1] = bb_hi
    pca_ref[:, _D + 1:] = bb_lo


def _dist_loss_kernel(z_ref, g_ref, pca_ref, ze_ref, ge_ref, mp_ref,
                      ft_sum_ref, mp_sum_ref, rhs_scr):
    i = pl.program_id(0)
    first_half = i < _NI_HALF

    a = jnp.where(first_half, z_ref[...], g_ref[...])
    at = jnp.swapaxes(a, 0, 1)
    aa = jnp.sum(at * at, axis=0, keepdims=True)

    rhs_scr[:_D, :] = (-2.0 * at).astype(jnp.bfloat16)
    rhs_scr[_D:, :] = jnp.ones((2, _BI), jnp.bfloat16)
    rhs = rhs_scr[...]

    pm = None
    for j in range(_NJ):
        ab = jax.lax.dot_general(
            pca_ref[j * _BJ:(j + 1) * _BJ, :], rhs,
            dimension_numbers=(((1,), (0,)), ((), ())),
            preferred_element_type=jnp.float32)
        m = jnp.min(ab, axis=0, keepdims=True)
        pm = m if pm is None else jnp.minimum(pm, m)

    d = pm + aa
    pe = jnp.where(first_half, ze_ref[...], ge_ref[...])
    pet = jnp.swapaxes(pe, 0, 1)
    ft = jnp.abs(jnp.sqrt(jnp.abs(d) + 1e-7) - jnp.abs(pet))
    ft_blk = jnp.sum(ft)
    mp_blk = jnp.sum(jnp.abs(mp_ref[...]))

    @pl.when(i == 0)
    def _():
        ft_sum_ref[0, 0] = ft_blk
        mp_sum_ref[0, 0] = mp_blk

    @pl.when(i > 0)
    def _():
        ft_sum_ref[0, 0] += ft_blk
        mp_sum_ref[0, 0] += mp_blk


def _half_map(i):
    return (jnp.minimum(i, _NI_HALF - 1), 0)


def _gen_map(i):
    return (jnp.maximum(i - _NI_HALF, 0), 0)


@functools.partial(jax.jit, static_argnames=("interpret",))
def _dist_loss(zero_pts, gen_pts, pc, zero_eval, gen_eval, manifold,
               interpret=False):
    pca = pl.pallas_call(
        _build_keys_kernel,
        out_shape=jax.ShapeDtypeStruct((_N_PC, _DA), jnp.bfloat16),
        interpret=interpret,
    )(pc)

    ft_sum, mp_sum = pl.pallas_call(
        _dist_loss_kernel,
        grid=(_NI,),
        in_specs=[
            pl.BlockSpec((_BI, _D), _half_map),
            pl.BlockSpec((_BI, _D), _gen_map),
            pl.BlockSpec((_N_PC, _DA), lambda i: (0, 0)),
            pl.BlockSpec((_BI, 1), _half_map),
            pl.BlockSpec((_BI, 1), _gen_map),
            pl.BlockSpec((_BI, 1), lambda i: (i, 0)),
        ],
        out_specs=[
            pl.BlockSpec(memory_space=pltpu.SMEM),
            pl.BlockSpec(memory_space=pltpu.SMEM),
        ],
        out_shape=[
            jax.ShapeDtypeStruct((1, 1), jnp.float32),
            jax.ShapeDtypeStruct((1, 1), jnp.float32),
        ],
        scratch_shapes=[
            pltpu.VMEM((_DA, _BI), jnp.bfloat16),
        ],
        compiler_params=pltpu.CompilerParams(
            dimension_semantics=("arbitrary",),
        ),
        interpret=interpret,
    )(zero_pts, gen_pts, pca, zero_eval, gen_eval, manifold)
    return ft_sum[0, 0], mp_sum[0, 0]


def kernel(zerolevelset_points, genlevelset_points, pc_input,
           zerolevelset_eval, gen_points_eval, manifold_pnts_pred,
           loss_lambda):
    ft_sum, mp_sum = _dist_loss(
        zerolevelset_points, genlevelset_points, pc_input,
        zerolevelset_eval, gen_points_eval, manifold_pnts_pred)

    mean_first = ft_sum / _N_PROJ
    mean_second = mp_sum / _N_PROJ
    ll = 0.1 if loss_lambda is None else loss_lambda
    loss = mean_first + ll * mean_second
    return (loss, mean_first, mean_second)


# speedup vs baseline: 1.1410x; 1.0557x over previous
import functools

import jax
import jax.numpy as jnp
from jax.experimental import pallas as pl
from jax.experimental.pallas import tpu as pltpu


_N_HALF = 4096
_N_PROJ = 8192
_N_PC = 8192
_D = 128
_DA = _D + 2

_BI = 2048
_NI = _N_PROJ // _BI
_NI_HALF = _N_HALF // _BI


def _dist_loss_kernel(z_ref, g_ref, pc_ref, ze_ref, ge_ref, mp_ref,
                      ft_sum_ref, mp_sum_ref, pca_scr, rhs_scr):
    i = pl.program_id(0)
    first_half = i < _NI_HALF

    @pl.when(i == 0)
    def _():
        pc = pc_ref[...]
        bb = jnp.sum(pc * pc, axis=1, keepdims=True)
        bb_hi = bb.astype(jnp.bfloat16)
        bb_lo = (bb - bb_hi.astype(jnp.float32)).astype(jnp.bfloat16)
        pca_scr[:, :_D] = pc.astype(jnp.bfloat16)
        pca_scr[:, _D:_D + 1] = bb_hi
        pca_scr[:, _D + 1:] = bb_lo

    a = jnp.where(first_half, z_ref[...], g_ref[...])
    at = jnp.swapaxes(a, 0, 1)
    aa = jnp.sum(at * at, axis=0, keepdims=True)

    rhs_scr[:_D, :] = (-2.0 * at).astype(jnp.bfloat16)
    rhs_scr[_D:, :] = jnp.ones((2, _BI), jnp.bfloat16)
    rhs = rhs_scr[...]

    ab = jax.lax.dot_general(
        pca_scr[...], rhs,
        dimension_numbers=(((1,), (0,)), ((), ())),
        preferred_element_type=jnp.float32)
    pm = jnp.min(ab, axis=0, keepdims=True)

    d = pm + aa
    pe = jnp.where(first_half, ze_ref[...], ge_ref[...])
    pet = jnp.swapaxes(pe, 0, 1)
    ft = jnp.abs(jnp.sqrt(jnp.abs(d) + 1e-7) - jnp.abs(pet))
    ft_blk = jnp.sum(ft)
    mp_blk = jnp.sum(jnp.abs(mp_ref[...]))

    @pl.when(i == 0)
    def _():
        ft_sum_ref[0, 0] = ft_blk
        mp_sum_ref[0, 0] = mp_blk

    @pl.when(i > 0)
    def _():
        ft_sum_ref[0, 0] += ft_blk
        mp_sum_ref[0, 0] += mp_blk


def _half_map(i):
    return (jnp.minimum(i, _NI_HALF - 1), 0)


def _gen_map(i):
    return (jnp.maximum(i - _NI_HALF, 0), 0)


@functools.partial(jax.jit, static_argnames=("interpret",))
def _dist_loss(zero_pts, gen_pts, pc, zero_eval, gen_eval, manifold,
               interpret=False):
    ft_sum, mp_sum = pl.pallas_call(
        _dist_loss_kernel,
        grid=(_NI,),
        in_specs=[
            pl.BlockSpec((_BI, _D), _half_map),
            pl.BlockSpec((_BI, _D), _gen_map),
            pl.BlockSpec((_N_PC, _D), lambda i: (0, 0)),
            pl.BlockSpec((_BI, 1), _half_map),
            pl.BlockSpec((_BI, 1), _gen_map),
            pl.BlockSpec((_BI, 1), lambda i: (i, 0)),
        ],
        out_specs=[
            pl.BlockSpec(memory_space=pltpu.SMEM),
            pl.BlockSpec(memory_space=pltpu.SMEM),
        ],
        out_shape=[
            jax.ShapeDtypeStruct((1, 1), jnp.float32),
            jax.ShapeDtypeStruct((1, 1), jnp.float32),
        ],
        scratch_shapes=[
            pltpu.VMEM((_N_PC, _DA), jnp.bfloat16),
            pltpu.VMEM((_DA, _BI), jnp.bfloat16),
        ],
        compiler_params=pltpu.CompilerParams(
            dimension_semantics=("arbitrary",),
        ),
        interpret=interpret,
    )(zero_pts, gen_pts, pc, zero_eval, gen_eval, manifold)
    return ft_sum[0, 0], mp_sum[0, 0]


def kernel(zerolevelset_points, genlevelset_points, pc_input,
           zerolevelset_eval, gen_points_eval, manifold_pnts_pred,
           loss_lambda):
    ft_sum, mp_sum = _dist_loss(
        zerolevelset_points, genlevelset_points, pc_input,
        zerolevelset_eval, gen_points_eval, manifold_pnts_pred)

    mean_first = ft_sum / _N_PROJ
    mean_second = mp_sum / _N_PROJ
    ll = 0.1 if loss_lambda is None else loss_lambda
    loss = mean_first + ll * mean_second
    return (loss, mean_first, mean_second)


# speedup vs baseline: 1.3441x; 1.1780x over previous
import functools

import jax
import jax.numpy as jnp
from jax.experimental import pallas as pl
from jax.experimental.pallas import tpu as pltpu


_N_HALF = 4096
_N_PROJ = 8192
_N_PC = 8192
_D = 128
_DA = _D + 2

_BI = 2048
_NI = _N_PROJ // _BI
_NI_HALF = _N_HALF // _BI


def _dist_loss_kernel(z_ref, g_ref, pc_ref, ze_ref, ge_ref, mp_ref,
                      ft_sum_ref, mp_sum_ref, pca_scr, rhs_scr):
    i = pl.program_id(0)
    first_half = i < _NI_HALF

    @pl.when(i == 0)
    def _():
        pc = pc_ref[...]
        bb = jnp.sum(pc * pc, axis=1, keepdims=True)
        bb_hi = bb.astype(jnp.bfloat16)
        bb_lo = (bb - bb_hi.astype(jnp.float32)).astype(jnp.bfloat16)
        pca_scr[:, :_D] = pc.astype(jnp.bfloat16)
        pca_scr[:, _D:_D + 1] = bb_hi
        pca_scr[:, _D + 1:] = bb_lo

    a = jnp.where(first_half, z_ref[...], g_ref[...])
    at = jnp.swapaxes(a, 0, 1)
    aa = jnp.sum(at * at, axis=0, keepdims=True)

    rhs_scr[:_D, :] = (-2.0 * at).astype(jnp.bfloat16)
    rhs_scr[_D:, :] = jnp.ones((2, _BI), jnp.bfloat16)
    rhs = rhs_scr[...]

    ab = jax.lax.dot_general(
        pca_scr[...], rhs,
        dimension_numbers=(((1,), (0,)), ((), ())),
        preferred_element_type=jnp.float32)
    pm = jnp.min(ab, axis=0, keepdims=True)

    d = jnp.reshape(pm + aa, (_BI // 128, 128))
    pe = jnp.where(first_half, ze_ref[...], ge_ref[...])
    ft = jnp.abs(jnp.sqrt(jnp.abs(d) + 1e-7) - jnp.abs(pe))
    ft_blk = jnp.sum(ft)
    mp_blk = jnp.sum(jnp.abs(mp_ref[...]))

    @pl.when(i == 0)
    def _():
        ft_sum_ref[0, 0] = ft_blk
        mp_sum_ref[0, 0] = mp_blk

    @pl.when(i > 0)
    def _():
        ft_sum_ref[0, 0] += ft_blk
        mp_sum_ref[0, 0] += mp_blk


def _half_map(i):
    return (jnp.minimum(i, _NI_HALF - 1), 0)


def _gen_map(i):
    return (jnp.maximum(i - _NI_HALF, 0), 0)


@functools.partial(jax.jit, static_argnames=("interpret",))
def _dist_loss(zero_pts, gen_pts, pc, zero_eval, gen_eval, manifold,
               interpret=False):
    ft_sum, mp_sum = pl.pallas_call(
        _dist_loss_kernel,
        grid=(_NI,),
        in_specs=[
            pl.BlockSpec((_BI, _D), _half_map),
            pl.BlockSpec((_BI, _D), _gen_map),
            pl.BlockSpec((_N_PC, _D), lambda i: (0, 0)),
            pl.BlockSpec((_BI // 128, 128), _half_map),
            pl.BlockSpec((_BI // 128, 128), _gen_map),
            pl.BlockSpec((_BI // 128, 128), lambda i: (i, 0)),
        ],
        out_specs=[
            pl.BlockSpec(memory_space=pltpu.SMEM),
            pl.BlockSpec(memory_space=pltpu.SMEM),
        ],
        out_shape=[
            jax.ShapeDtypeStruct((1, 1), jnp.float32),
            jax.ShapeDtypeStruct((1, 1), jnp.float32),
        ],
        scratch_shapes=[
            pltpu.VMEM((_N_PC, _DA), jnp.bfloat16),
            pltpu.VMEM((_DA, _BI), jnp.bfloat16),
        ],
        compiler_params=pltpu.CompilerParams(
            dimension_semantics=("arbitrary",),
        ),
        interpret=interpret,
    )(zero_pts, gen_pts, pc,
      jnp.reshape(zero_eval, (_N_HALF // 128, 128)),
      jnp.reshape(gen_eval, (_N_HALF // 128, 128)),
      jnp.reshape(manifold, (_N_PROJ // 128, 128)))
    return ft_sum[0, 0], mp_sum[0, 0]


def kernel(zerolevelset_points, genlevelset_points, pc_input,
           zerolevelset_eval, gen_points_eval, manifold_pnts_pred,
           loss_lambda):
    ft_sum, mp_sum = _dist_loss(
        zerolevelset_points, genlevelset_points, pc_input,
        zerolevelset_eval, gen_points_eval, manifold_pnts_pred)

    mean_first = ft_sum / _N_PROJ
    mean_second = mp_sum / _N_PROJ
    ll = 0.1 if loss_lambda is None else loss_lambda
    loss = mean_first + ll * mean_second
    return (loss, mean_first, mean_second)


# speedup vs baseline: 1.3531x; 1.0067x over previous
import functools

import jax
import jax.numpy as jnp
from jax.experimental import pallas as pl
from jax.experimental.pallas import tpu as pltpu


_N_HALF = 4096
_N_PROJ = 8192
_N_PC = 8192
_D = 128
_DA = _D + 2

_BI = 4096
_NI = _N_PROJ // _BI
_NI_HALF = _N_HALF // _BI


def _dist_loss_kernel(z_ref, g_ref, pc_ref, ze_ref, ge_ref, mp_ref,
                      ft_sum_ref, mp_sum_ref, pca_scr, rhs_scr):
    i = pl.program_id(0)
    first_half = i < _NI_HALF

    @pl.when(i == 0)
    def _():
        pc = pc_ref[...]
        bb = jnp.sum(pc * pc, axis=1, keepdims=True)
        bb_hi = bb.astype(jnp.bfloat16)
        bb_lo = (bb - bb_hi.astype(jnp.float32)).astype(jnp.bfloat16)
        pca_scr[:, :_D] = pc.astype(jnp.bfloat16)
        pca_scr[:, _D:_D + 1] = bb_hi
        pca_scr[:, _D + 1:] = bb_lo

    a = jnp.where(first_half, z_ref[...], g_ref[...])
    at = jnp.swapaxes(a, 0, 1)
    aa = jnp.sum(at * at, axis=0, keepdims=True)

    rhs_scr[:_D, :] = (-2.0 * at).astype(jnp.bfloat16)
    rhs_scr[_D:, :] = jnp.ones((2, _BI), jnp.bfloat16)
    rhs = rhs_scr[...]

    ab = jax.lax.dot_general(
        pca_scr[...], rhs,
        dimension_numbers=(((1,), (0,)), ((), ())),
        preferred_element_type=jnp.float32)
    pm = jnp.min(ab, axis=0, keepdims=True)

    d = jnp.reshape(pm + aa, (_BI // 128, 128))
    pe = jnp.where(first_half, ze_ref[...], ge_ref[...])
    ft = jnp.abs(jnp.sqrt(jnp.abs(d) + 1e-7) - jnp.abs(pe))
    ft_blk = jnp.sum(ft)
    mp_blk = jnp.sum(jnp.abs(mp_ref[...]))

    @pl.when(i == 0)
    def _():
        ft_sum_ref[0, 0] = ft_blk
        mp_sum_ref[0, 0] = mp_blk

    @pl.when(i > 0)
    def _():
        ft_sum_ref[0, 0] += ft_blk
        mp_sum_ref[0, 0] += mp_blk


def _half_map(i):
    return (jnp.minimum(i, _NI_HALF - 1), 0)


def _gen_map(i):
    return (jnp.maximum(i - _NI_HALF, 0), 0)


@functools.partial(jax.jit, static_argnames=("interpret",))
def _dist_loss(zero_pts, gen_pts, pc, zero_eval, gen_eval, manifold,
               interpret=False):
    ft_sum, mp_sum = pl.pallas_call(
        _dist_loss_kernel,
        grid=(_NI,),
        in_specs=[
            pl.BlockSpec((_BI, _D), _half_map),
            pl.BlockSpec((_BI, _D), _gen_map),
            pl.BlockSpec((_N_PC, _D), lambda i: (0, 0)),
            pl.BlockSpec((_BI // 128, 128), _half_map),
            pl.BlockSpec((_BI // 128, 128), _gen_map),
            pl.BlockSpec((_BI // 128, 128), lambda i: (i, 0)),
        ],
        out_specs=[
            pl.BlockSpec(memory_space=pltpu.SMEM),
            pl.BlockSpec(memory_space=pltpu.SMEM),
        ],
        out_shape=[
            jax.ShapeDtypeStruct((1, 1), jnp.float32),
            jax.ShapeDtypeStruct((1, 1), jnp.float32),
        ],
        scratch_shapes=[
            pltpu.VMEM((_N_PC, _DA), jnp.bfloat16),
            pltpu.VMEM((_DA, _BI), jnp.bfloat16),
        ],
        compiler_params=pltpu.CompilerParams(
            dimension_semantics=("arbitrary",),
        ),
        interpret=interpret,
    )(zero_pts, gen_pts, pc,
      jnp.reshape(zero_eval, (_N_HALF // 128, 128)),
      jnp.reshape(gen_eval, (_N_HALF // 128, 128)),
      jnp.reshape(manifold, (_N_PROJ // 128, 128)))
    return ft_sum[0, 0], mp_sum[0, 0]


def kernel(zerolevelset_points, genlevelset_points, pc_input,
           zerolevelset_eval, gen_points_eval, manifold_pnts_pred,
           loss_lambda):
    ft_sum, mp_sum = _dist_loss(
        zerolevelset_points, genlevelset_points, pc_input,
        zerolevelset_eval, gen_points_eval, manifold_pnts_pred)

    mean_first = ft_sum / _N_PROJ
    mean_second = mp_sum / _N_PROJ
    ll = 0.1 if loss_lambda is None else loss_lambda
    loss = mean_first + ll * mean_second
    return (loss, mean_first, mean_second)
